# baseline (device time: 1171228 ns/iter reference)
import jax
import jax.numpy as jnp
from jax import lax
from jax.experimental import pallas as pl
from jax.experimental.pallas import tpu as pltpu

N_DEV = 32
B_PER = 2
SQ = 128
SKV = 128
H_PER = 4
DH = 64
D_MODEL = 512
DHEADS = H_PER * DH


def kernel(x, Wq, K_ext, V_ext, Wo):
    def body(x_ref, wq_ref, k_hbm, v_hbm, wo_ref, out_ref,
             pay_r, pay_l, kbuf_r, vbuf_r, kbuf_l, vbuf_l,
             send_r, recv_r, send_l, recv_l, kv_sems):
        my = lax.axis_index("i")
        left = (my - 1) % N_DEV
        right = (my + 1) % N_DEV

        barrier = pltpu.get_barrier_semaphore()
        for nbr in (left, right):
            pl.semaphore_signal(barrier, inc=1, device_id=(nbr,),
                                device_id_type=pl.DeviceIdType.MESH)
        pl.semaphore_wait(barrier, 2)

        wq = wq_ref[...]
        wo = wo_ref[...]

        def fetch_kv(gb, kbuf, vbuf):
            copies = [
                pltpu.make_async_copy(
                    k_hbm.at[gb, :, pl.ds(my * H_PER, H_PER), :],
                    kbuf, kv_sems.at[0]),
                pltpu.make_async_copy(
                    v_hbm.at[gb, :, pl.ds(my * H_PER, H_PER), :],
                    vbuf, kv_sems.at[1]),
            ]
            for cp in copies:
                cp.start()
            return copies

        def blockdiag(b3):
            rows = []
            for j in range(H_PER):
                pieces = []
                if j > 0:
                    pieces.append(jnp.zeros((SKV, j * DH), jnp.float32))
                pieces.append(b3[:, j, :])
                if j < H_PER - 1:
                    pieces.append(
                        jnp.zeros((SKV, (H_PER - 1 - j) * DH), jnp.float32))
                rows.append(jnp.concatenate(pieces, axis=1))
            return jnp.concatenate(rows, axis=0)

        def contrib(x_plane, kbuf, vbuf):
            kbd = blockdiag(kbuf[...])
            vbd = blockdiag(vbuf[...])
            q = jnp.dot(x_plane, wq, preferred_element_type=jnp.float32)
            s = lax.dot_general(
                q, kbd,
                (((1,), (1,)), ((), ())),
                preferred_element_type=jnp.float32)
            e = jnp.exp(s * 0.125)
            ws = []
            for h in range(H_PER):
                eh = e[:, h * SKV:(h + 1) * SKV]
                den = jnp.sum(eh, axis=-1, keepdims=True)
                ws.append(eh / den)
            w = jnp.concatenate(ws, axis=1)
            ctx = jnp.dot(w, vbd, preferred_element_type=jnp.float32)
            return jnp.dot(ctx, wo, preferred_element_type=jnp.float32)

        pay_r[0, 0] = x_ref[0]
        pay_l[0, 0] = x_ref[1]

        for t in range(N_DEV):
            ss = t % 2
            rs = (t + 1) % 2
            rdma_r = pltpu.make_async_remote_copy(
                src_ref=pay_r.at[ss], dst_ref=pay_r.at[rs],
                send_sem=send_r.at[ss], recv_sem=recv_r.at[rs],
                device_id=(right,), device_id_type=pl.DeviceIdType.MESH)
            rdma_l = pltpu.make_async_remote_copy(
                src_ref=pay_l.at[ss], dst_ref=pay_l.at[rs],
                send_sem=send_l.at[ss], recv_sem=recv_l.at[rs],
                device_id=(left,), device_id_type=pl.DeviceIdType.MESH)
            rdma_r.start()
            rdma_l.start()

            c_r = (my - t - 1) % N_DEV
            c_l = (my + t + 1) % N_DEV
            kv = fetch_kv(c_r * B_PER, kbuf_r, vbuf_r)
            kv += fetch_kv(c_l * B_PER + 1, kbuf_l, vbuf_l)

            rdma_r.wait()
            rdma_l.wait()
            for cp in kv:
                cp.wait()

            if t < N_DEV - 1:
                pr = contrib(pay_r[rs, 0], kbuf_r, vbuf_r)
                pl_ = contrib(pay_l[rs, 0], kbuf_l, vbuf_l)
                if t == 0:
                    pay_r[rs, 1] = pr
                    pay_l[rs, 1] = pl_
                else:
                    pay_r[rs, 1] = pay_r[rs, 1] + pr
                    pay_l[rs, 1] = pay_l[rs, 1] + pl_
            else:
                out_ref[0] = pay_r[rs, 1] + contrib(x_ref[0], kbuf_r, vbuf_r)
                out_ref[1] = pay_l[rs, 1] + contrib(x_ref[1], kbuf_l, vbuf_l)

    return pl.pallas_call(
        body,
        out_shape=jax.ShapeDtypeStruct((B_PER, SQ, D_MODEL), jnp.float32),
        in_specs=[
            pl.BlockSpec(memory_space=pltpu.VMEM),
            pl.BlockSpec(memory_space=pltpu.VMEM),
            pl.BlockSpec(memory_space=pl.ANY),
            pl.BlockSpec(memory_space=pl.ANY),
            pl.BlockSpec(memory_space=pltpu.VMEM),
        ],
        out_specs=pl.BlockSpec(memory_space=pltpu.VMEM),
        scratch_shapes=[
            pltpu.VMEM((2, 2, SQ, D_MODEL), jnp.float32),
            pltpu.VMEM((2, 2, SQ, D_MODEL), jnp.float32),
            pltpu.VMEM((SKV, H_PER, DH), jnp.float32),
            pltpu.VMEM((SKV, H_PER, DH), jnp.float32),
            pltpu.VMEM((SKV, H_PER, DH), jnp.float32),
            pltpu.VMEM((SKV, H_PER, DH), jnp.float32),
            pltpu.SemaphoreType.DMA((2,)),
            pltpu.SemaphoreType.DMA((2,)),
            pltpu.SemaphoreType.DMA((2,)),
            pltpu.SemaphoreType.DMA((2,)),
            pltpu.SemaphoreType.DMA((2,)),
        ],
        compiler_params=pltpu.CompilerParams(collective_id=0),
    )(x, Wq, K_ext, V_ext, Wo)


# device time: 1079447 ns/iter; 1.0850x vs baseline; 1.0850x over previous
import jax
import jax.numpy as jnp
from jax import lax
from jax.experimental import pallas as pl
from jax.experimental.pallas import tpu as pltpu

N_DEV = 32
B_PER = 2
B_GLB = 64
SQ = 128
SKV = 128
H_PER = 4
DH = 64
D_MODEL = 512
DHEADS = H_PER * DH


def kernel(x, Wq, K_ext, V_ext, Wo):
    idx = lax.axis_index("i")
    K_loc = lax.dynamic_slice(
        K_ext, (0, 0, idx * H_PER, 0), (B_GLB, SKV, H_PER, DH))
    V_loc = lax.dynamic_slice(
        V_ext, (0, 0, idx * H_PER, 0), (B_GLB, SKV, H_PER, DH))

    def body(x_ref, wq_ref, k_ref, v_ref, wo_ref, out_ref,
             pay_r, pay_l, send_r, recv_r, send_l, recv_l):
        my = lax.axis_index("i")
        left = (my - 1) % N_DEV
        right = (my + 1) % N_DEV

        barrier = pltpu.get_barrier_semaphore()
        for nbr in (left, right):
            pl.semaphore_signal(barrier, inc=1, device_id=(nbr,),
                                device_id_type=pl.DeviceIdType.MESH)
        pl.semaphore_wait(barrier, 2)

        wq = wq_ref[...]
        wo = wo_ref[...]

        def blockdiag(b3):
            rows = []
            for j in range(H_PER):
                pieces = []
                if j > 0:
                    pieces.append(jnp.zeros((SKV, j * DH), jnp.float32))
                pieces.append(b3[:, j, :])
                if j < H_PER - 1:
                    pieces.append(
                        jnp.zeros((SKV, (H_PER - 1 - j) * DH), jnp.float32))
                rows.append(jnp.concatenate(pieces, axis=1))
            return jnp.concatenate(rows, axis=0)

        def contrib(x_plane, gb):
            kbd = blockdiag(k_ref[gb])
            vbd = blockdiag(v_ref[gb])
            q = jnp.dot(x_plane, wq, preferred_element_type=jnp.float32)
            s = lax.dot_general(
                q, kbd,
                (((1,), (1,)), ((), ())),
                preferred_element_type=jnp.float32)
            e = jnp.exp(s * 0.125)
            ws = []
            for h in range(H_PER):
                eh = e[:, h * SKV:(h + 1) * SKV]
                den = jnp.sum(eh, axis=-1, keepdims=True)
                ws.append(eh / den)
            w = jnp.concatenate(ws, axis=1)
            ctx = jnp.dot(w, vbd, preferred_element_type=jnp.float32)
            return jnp.dot(ctx, wo, preferred_element_type=jnp.float32)

        pay_r[0, 0] = x_ref[0]
        pay_l[0, 0] = x_ref[1]

        for t in range(N_DEV):
            ss = t % 2
            rs = (t + 1) % 2
            rdma_r = pltpu.make_async_remote_copy(
                src_ref=pay_r.at[ss], dst_ref=pay_r.at[rs],
                send_sem=send_r.at[ss], recv_sem=recv_r.at[rs],
                device_id=(right,), device_id_type=pl.DeviceIdType.MESH)
            rdma_l = pltpu.make_async_remote_copy(
                src_ref=pay_l.at[ss], dst_ref=pay_l.at[rs],
                send_sem=send_l.at[ss], recv_sem=recv_l.at[rs],
                device_id=(left,), device_id_type=pl.DeviceIdType.MESH)
            rdma_r.start()
            rdma_l.start()

            c_r = (my - t - 1) % N_DEV
            c_l = (my + t + 1) % N_DEV

            rdma_r.wait()
            rdma_l.wait()

            if t < N_DEV - 1:
                pr = contrib(pay_r[rs, 0], c_r * B_PER)
                pl_ = contrib(pay_l[rs, 0], c_l * B_PER + 1)
                if t == 0:
                    pay_r[rs, 1] = pr
                    pay_l[rs, 1] = pl_
                else:
                    pay_r[rs, 1] = pay_r[rs, 1] + pr
                    pay_l[rs, 1] = pay_l[rs, 1] + pl_
            else:
                out_ref[0] = pay_r[rs, 1] + contrib(x_ref[0], my * B_PER)
                out_ref[1] = pay_l[rs, 1] + contrib(x_ref[1], my * B_PER + 1)

    return pl.pallas_call(
        body,
        out_shape=jax.ShapeDtypeStruct((B_PER, SQ, D_MODEL), jnp.float32),
        in_specs=[
            pl.BlockSpec(memory_space=pltpu.VMEM),
            pl.BlockSpec(memory_space=pltpu.VMEM),
            pl.BlockSpec(memory_space=pltpu.VMEM),
            pl.BlockSpec(memory_space=pltpu.VMEM),
            pl.BlockSpec(memory_space=pltpu.VMEM),
        ],
        out_specs=pl.BlockSpec(memory_space=pltpu.VMEM),
        scratch_shapes=[
            pltpu.VMEM((2, 2, SQ, D_MODEL), jnp.float32),
            pltpu.VMEM((2, 2, SQ, D_MODEL), jnp.float32),
            pltpu.SemaphoreType.DMA((2,)),
            pltpu.SemaphoreType.DMA((2,)),
            pltpu.SemaphoreType.DMA((2,)),
            pltpu.SemaphoreType.DMA((2,)),
        ],
        compiler_params=pltpu.CompilerParams(collective_id=0),
    )(x, Wq, K_loc, V_loc, Wo)


# device time: 469316 ns/iter; 2.4956x vs baseline; 2.3000x over previous
import jax
import jax.numpy as jnp
from jax import lax
from jax.experimental import pallas as pl
from jax.experimental.pallas import tpu as pltpu

N_DEV = 32
B_PER = 2
HQ = 128
SQ = 128
SKV = 128
H_PER = 4
DH = 64
D_MODEL = 512
DHEADS = H_PER * DH


def kernel(x, Wq, K_ext, V_ext, Wo):
    idx = lax.axis_index("i")
    Ks = lax.dynamic_slice(
        K_ext, (idx * B_PER, 0, 0, 0), (B_PER, SKV, HQ, DH))
    Vs = lax.dynamic_slice(
        V_ext, (idx * B_PER, 0, 0, 0), (B_PER, SKV, HQ, DH))
    Kt = jnp.transpose(Ks, (0, 2, 1, 3)).reshape(B_PER * HQ, SKV, DH)
    Vt = jnp.transpose(Vs, (0, 2, 1, 3)).reshape(B_PER * HQ, SKV, DH)

    def body(x_ref, wq_ref, k_ref, v_ref, wo_ref, out_ref,
             bq_r, bo_r, bq_l, bo_l,
             sq_r, rq_r, so_r, ro_r, sq_l, rq_l, so_l, ro_l):
        my = lax.axis_index("i")
        left = (my - 1) % N_DEV
        right = (my + 1) % N_DEV

        barrier = pltpu.get_barrier_semaphore()
        for nbr in (left, right):
            pl.semaphore_signal(barrier, inc=1, device_id=(nbr,),
                                device_id_type=pl.DeviceIdType.MESH)
        pl.semaphore_wait(barrier, 2)

        def blockdiag(ref, b, j):
            b3 = ref[pl.ds(b * HQ + j * H_PER, H_PER)]
            rows = []
            for h in range(H_PER):
                pieces = []
                if h > 0:
                    pieces.append(jnp.zeros((SKV, h * DH), jnp.float32))
                pieces.append(b3[h])
                if h < H_PER - 1:
                    pieces.append(
                        jnp.zeros((SKV, (H_PER - 1 - h) * DH), jnp.float32))
                rows.append(jnp.concatenate(pieces, axis=1))
            return jnp.concatenate(rows, axis=0)

        def add_contrib(j, wqj, woj, first):
            for b in range(B_PER):
                kbd = blockdiag(k_ref, b, j)
                vbd = blockdiag(v_ref, b, j)
                q = jnp.dot(x_ref[b], wqj,
                            preferred_element_type=jnp.float32)
                s = lax.dot_general(
                    q, kbd,
                    (((1,), (1,)), ((), ())),
                    preferred_element_type=jnp.float32)
                e = jnp.exp(s * 0.125)
                ws = []
                for h in range(H_PER):
                    eh = e[:, h * SKV:(h + 1) * SKV]
                    den = jnp.sum(eh, axis=-1, keepdims=True)
                    ws.append(eh / den)
                w = jnp.concatenate(ws, axis=1)
                ctx = jnp.dot(w, vbd, preferred_element_type=jnp.float32)
                part = jnp.dot(ctx, woj, preferred_element_type=jnp.float32)
                if first:
                    out_ref[b] = part
                else:
                    out_ref[b] = out_ref[b] + part

        wq_own = wq_ref[...]
        wo_own = wo_ref[...]
        bq_r[0] = wq_own
        bo_r[0] = wo_own
        bq_l[0] = wq_own
        bo_l[0] = wo_own

        for t in range(16):
            ss = t % 4
            rs = (t + 1) % 4
            rdmas = [
                pltpu.make_async_remote_copy(
                    src_ref=bq_r.at[ss], dst_ref=bq_r.at[rs],
                    send_sem=sq_r.at[ss], recv_sem=rq_r.at[rs],
                    device_id=(right,), device_id_type=pl.DeviceIdType.MESH),
                pltpu.make_async_remote_copy(
                    src_ref=bo_r.at[ss], dst_ref=bo_r.at[rs],
                    send_sem=so_r.at[ss], recv_sem=ro_r.at[rs],
                    device_id=(right,), device_id_type=pl.DeviceIdType.MESH),
            ]
            if t < 15:
                rdmas += [
                    pltpu.make_async_remote_copy(
                        src_ref=bq_l.at[ss], dst_ref=bq_l.at[rs],
                        send_sem=sq_l.at[ss], recv_sem=rq_l.at[rs],
                        device_id=(left,),
                        device_id_type=pl.DeviceIdType.MESH),
                    pltpu.make_async_remote_copy(
                        src_ref=bo_l.at[ss], dst_ref=bo_l.at[rs],
                        send_sem=so_l.at[ss], recv_sem=ro_l.at[rs],
                        device_id=(left,),
                        device_id_type=pl.DeviceIdType.MESH),
                ]
            for r in rdmas:
                r.start()

            if t == 0:
                add_contrib(my, wq_own, wo_own, first=True)

            for r in rdmas:
                r.wait()

            j_r = (my - t - 1) % N_DEV
            add_contrib(j_r, bq_r[rs], bo_r[rs], first=False)
            if t < 15:
                j_l = (my + t + 1) % N_DEV
                add_contrib(j_l, bq_l[rs], bo_l[rs], first=False)

    return pl.pallas_call(
        body,
        out_shape=jax.ShapeDtypeStruct((B_PER, SQ, D_MODEL), jnp.float32),
        in_specs=[
            pl.BlockSpec(memory_space=pltpu.VMEM),
            pl.BlockSpec(memory_space=pltpu.VMEM),
            pl.BlockSpec(memory_space=pltpu.VMEM),
            pl.BlockSpec(memory_space=pltpu.VMEM),
            pl.BlockSpec(memory_space=pltpu.VMEM),
        ],
        out_specs=pl.BlockSpec(memory_space=pltpu.VMEM),
        scratch_shapes=[
            pltpu.VMEM((4, D_MODEL, DHEADS), jnp.float32),
            pltpu.VMEM((4, DHEADS, D_MODEL), jnp.float32),
            pltpu.VMEM((4, D_MODEL, DHEADS), jnp.float32),
            pltpu.VMEM((4, DHEADS, D_MODEL), jnp.float32),
            pltpu.SemaphoreType.DMA((4,)),
            pltpu.SemaphoreType.DMA((4,)),
            pltpu.SemaphoreType.DMA((4,)),
            pltpu.SemaphoreType.DMA((4,)),
            pltpu.SemaphoreType.DMA((4,)),
            pltpu.SemaphoreType.DMA((4,)),
            pltpu.SemaphoreType.DMA((4,)),
            pltpu.SemaphoreType.DMA((4,)),
        ],
        compiler_params=pltpu.CompilerParams(collective_id=0),
    )(x, Wq, Kt, Vt, Wo)


# device time: 436897 ns/iter; 2.6808x vs baseline; 1.0742x over previous
import jax
import jax.numpy as jnp
from jax import lax
from jax.experimental import pallas as pl
from jax.experimental.pallas import tpu as pltpu

N_DEV = 32
B_PER = 2
HQ = 128
SQ = 128
SKV = 128
H_PER = 4
DH = 64
D_MODEL = 512
DHEADS = H_PER * DH


def kernel(x, Wq, K_ext, V_ext, Wo):
    idx = lax.axis_index("i")
    Ks = lax.dynamic_slice(
        K_ext, (idx * B_PER, 0, 0, 0), (B_PER, SKV, HQ, DH))
    Vs = lax.dynamic_slice(
        V_ext, (idx * B_PER, 0, 0, 0), (B_PER, SKV, HQ, DH))
    Kt = jnp.transpose(Ks, (0, 2, 1, 3)).reshape(B_PER * HQ, SKV, DH)
    Vt = jnp.transpose(Vs, (0, 2, 1, 3)).reshape(B_PER * HQ, SKV, DH)

    def body(x_ref, wq_ref, k_ref, v_ref, wo_ref, out_ref,
             bq_r, bo_r, bq_l, bo_l,
             sq_r, rq_r, so_r, ro_r, sq_l, rq_l, so_l, ro_l):
        my = lax.axis_index("i")
        left = (my - 1) % N_DEV
        right = (my + 1) % N_DEV

        barrier = pltpu.get_barrier_semaphore()
        for nbr in (left, right):
            pl.semaphore_signal(barrier, inc=1, device_id=(nbr,),
                                device_id_type=pl.DeviceIdType.MESH)
        pl.semaphore_wait(barrier, 2)

        def blockdiag(ref, b, j):
            b3 = ref[pl.ds(b * HQ + j * H_PER, H_PER)]
            rows = []
            for h in range(H_PER):
                pieces = []
                if h > 0:
                    pieces.append(jnp.zeros((SKV, h * DH), jnp.float32))
                pieces.append(b3[h])
                if h < H_PER - 1:
                    pieces.append(
                        jnp.zeros((SKV, (H_PER - 1 - h) * DH), jnp.float32))
                rows.append(jnp.concatenate(pieces, axis=1))
            return jnp.concatenate(rows, axis=0)

        def add_contrib(j, wqj, woj, first):
            for b in range(B_PER):
                kbd = blockdiag(k_ref, b, j)
                vbd = blockdiag(v_ref, b, j)
                q = jnp.dot(x_ref[b], wqj,
                            preferred_element_type=jnp.float32)
                s = lax.dot_general(
                    q, kbd,
                    (((1,), (1,)), ((), ())),
                    preferred_element_type=jnp.float32)
                e = jnp.exp(s * 0.125)
                ws = []
                for h in range(H_PER):
                    eh = e[:, h * SKV:(h + 1) * SKV]
                    den = jnp.sum(eh, axis=-1, keepdims=True)
                    ws.append(eh / den)
                w = jnp.concatenate(ws, axis=1)
                ctx = jnp.dot(w, vbd, preferred_element_type=jnp.float32)
                part = jnp.dot(ctx, woj, preferred_element_type=jnp.float32)
                if first:
                    out_ref[b] = part
                else:
                    out_ref[b] = out_ref[b] + part

        wq_own = wq_ref[...]
        wo_own = wo_ref[...]
        bq_r[0] = wq_own
        bo_r[0] = wo_own
        bq_l[0] = wq_own
        bo_l[0] = wo_own

        def make_rdmas(t):
            ss = t % 4
            rs = (t + 1) % 4
            rr = [(bq_r, sq_r, rq_r, right), (bo_r, so_r, ro_r, right)]
            if t < 15:
                rr += [(bq_l, sq_l, rq_l, left), (bo_l, so_l, ro_l, left)]
            return [
                pltpu.make_async_remote_copy(
                    src_ref=buf.at[ss], dst_ref=buf.at[rs],
                    send_sem=snd.at[ss], recv_sem=rcv.at[rs],
                    device_id=(dev,), device_id_type=pl.DeviceIdType.MESH)
                for buf, snd, rcv, dev in rr]

        rlists = {0: make_rdmas(0)}
        for r in rlists[0]:
            r.start()
        add_contrib(my, wq_own, wo_own, first=True)

        for t in range(16):
            rs = (t + 1) % 4
            for r in rlists[t]:
                r.wait_recv()
            if t < 15:
                if t >= 3:
                    for r in rlists[t - 3]:
                        r.wait_send()
                rlists[t + 1] = make_rdmas(t + 1)
                for r in rlists[t + 1]:
                    r.start()
            j_r = (my - t - 1) % N_DEV
            add_contrib(j_r, bq_r[rs], bo_r[rs], first=False)
            if t < 15:
                j_l = (my + t + 1) % N_DEV
                add_contrib(j_l, bq_l[rs], bo_l[rs], first=False)

        for tt in (12, 13, 14, 15):
            for r in rlists[tt]:
                r.wait_send()

    return pl.pallas_call(
        body,
        out_shape=jax.ShapeDtypeStruct((B_PER, SQ, D_MODEL), jnp.float32),
        in_specs=[
            pl.BlockSpec(memory_space=pltpu.VMEM),
            pl.BlockSpec(memory_space=pltpu.VMEM),
            pl.BlockSpec(memory_space=pltpu.VMEM),
            pl.BlockSpec(memory_space=pltpu.VMEM),
            pl.BlockSpec(memory_space=pltpu.VMEM),
        ],
        out_specs=pl.BlockSpec(memory_space=pltpu.VMEM),
        scratch_shapes=[
            pltpu.VMEM((4, D_MODEL, DHEADS), jnp.float32),
            pltpu.VMEM((4, DHEADS, D_MODEL), jnp.float32),
            pltpu.VMEM((4, D_MODEL, DHEADS), jnp.float32),
            pltpu.VMEM((4, DHEADS, D_MODEL), jnp.float32),
            pltpu.SemaphoreType.DMA((4,)),
            pltpu.SemaphoreType.DMA((4,)),
            pltpu.SemaphoreType.DMA((4,)),
            pltpu.SemaphoreType.DMA((4,)),
            pltpu.SemaphoreType.DMA((4,)),
            pltpu.SemaphoreType.DMA((4,)),
            pltpu.SemaphoreType.DMA((4,)),
            pltpu.SemaphoreType.DMA((4,)),
        ],
        compiler_params=pltpu.CompilerParams(collective_id=0),
    )(x, Wq, Kt, Vt, Wo)
